# baseline (speedup 1.0000x reference)
import jax
import jax.numpy as jnp
from jax import lax
from jax.experimental import pallas as pl

BB = 16


def _posenc_block(ts_ref, emb_ref, out_ref):
    t = ts_ref[...]
    bb, hist = t.shape
    expand = emb_ref.shape[1]
    tk = t[:, :, None]
    tj = t[:, None, :]
    kk2 = lax.broadcasted_iota(jnp.int32, (hist, hist), 0)
    jj2 = lax.broadcasted_iota(jnp.int32, (hist, hist), 1)
    tri = (kk2 < jj2)[None]
    masked = ((tk <= tj) & tri).astype(jnp.int32)
    colsum = jnp.sum(masked, axis=1)
    rowsum = jnp.sum(masked, axis=2)
    jpos = lax.broadcasted_iota(jnp.int32, (bb, hist), 1)
    rank = colsum + (hist - 1 - jpos) - rowsum
    ii = lax.broadcasted_iota(jnp.int32, (bb, hist, hist), 1)
    m = (rank[:, None, :] == ii).astype(jnp.float32)
    out = jnp.dot(m.reshape(bb * hist, hist), emb_ref[...],
                  preferred_element_type=jnp.float32)
    out_ref[...] = out.reshape(bb, hist, expand)


def kernel(ts, pos_embeddings):
    batch, hist = ts.shape
    seq_len, expand = pos_embeddings.shape
    return pl.pallas_call(
        _posenc_block,
        grid=(batch // BB,),
        in_specs=[
            pl.BlockSpec((BB, hist), lambda i: (i, 0)),
            pl.BlockSpec((seq_len, expand), lambda i: (0, 0)),
        ],
        out_specs=pl.BlockSpec((BB, hist, expand), lambda i: (i, 0, 0)),
        out_shape=jax.ShapeDtypeStruct((batch, hist, expand), jnp.float32),
    )(ts, pos_embeddings)


# speedup vs baseline: 1.2625x; 1.2625x over previous
import jax
import jax.numpy as jnp
from jax import lax
from jax.experimental import pallas as pl

BB = 16


def _posenc_block(ts_ref, emb_ref, out_ref):
    t = ts_ref[...]
    bb, hist = t.shape
    expand = emb_ref.shape[1]
    tk = t[:, :, None]
    tj = t[:, None, :]
    kk2 = lax.broadcasted_iota(jnp.int32, (hist, hist), 0)
    jj2 = lax.broadcasted_iota(jnp.int32, (hist, hist), 1)
    tri = (kk2 < jj2)[None]
    c = ((tk < tj) | ((tk <= tj) & tri)).astype(jnp.int32)
    rank = jnp.sum(c, axis=1)
    ii = lax.broadcasted_iota(jnp.int32, (bb, hist, hist), 1)
    m = (rank[:, None, :] == ii).astype(jnp.float32)
    out = jnp.dot(m.reshape(bb * hist, hist), emb_ref[...],
                  preferred_element_type=jnp.float32)
    out_ref[...] = out.reshape(bb, hist, expand)


def kernel(ts, pos_embeddings):
    batch, hist = ts.shape
    seq_len, expand = pos_embeddings.shape
    return pl.pallas_call(
        _posenc_block,
        grid=(batch // BB,),
        in_specs=[
            pl.BlockSpec((BB, hist), lambda i: (i, 0)),
            pl.BlockSpec((seq_len, expand), lambda i: (0, 0)),
        ],
        out_specs=pl.BlockSpec((BB, hist, expand), lambda i: (i, 0, 0)),
        out_shape=jax.ShapeDtypeStruct((batch, hist, expand), jnp.float32),
    )(ts, pos_embeddings)
